# baseline (speedup 1.0000x reference)
import functools

import jax
import jax.numpy as jnp
import numpy as np
from jax import lax
from jax.experimental import pallas as pl
from jax.experimental.pallas import tpu as pltpu
from jax.experimental.pallas import tpu_sc as plsc

_B = 128
_V = 100000
_L = 16
_NW = 32
_NBG = _B // _L
_VW = 3128
_W = 184
_TWR = _V - 31 * _VW - 16 * _W
_U1 = 4
_UT = 2

_NEG_INF = float("-inf")
_I32_BIG = np.int32(2**31 - 1)


def _sc_body(lt, act, g_out, m_out, s_out, i_out,
             buf_a, buf_b, buf_t, act_v, gst, mst, sst, ist,
             sem_a, sem_b, sem_t):
    w = lax.axis_index("s") * 2 + lax.axis_index("c")
    vb = w * _VW
    lane = lax.iota(jnp.int32, _L)
    neg = jnp.full((_L,), _NEG_INF, jnp.float32)
    zf = jnp.zeros((_L,), jnp.float32)
    zi = jnp.zeros((_L,), jnp.int32)

    pltpu.sync_copy(act, act_v)
    for bg in range(_NBG):
        mst[pl.ds(bg * _L, _L)] = neg
        ist[pl.ds(bg * _L, _L)] = zi
        sst[pl.ds(bg * _L, _L)] = zf
        gst[pl.ds(bg * _L, _L)] = zf

    def dma(c, buf, sem):
        pltpu.async_copy(lt.at[pl.ds(vb + c * _W, _W), :], buf, sem)

    def wait(buf, sem):
        pltpu.make_async_copy(lt.at[pl.ds(0, _W), :], buf, sem).wait()

    def process(buf, off, nv, un):
        m0 = [mst[pl.ds(bg * _L, _L)] for bg in range(_NBG)]

        def p1(i, car):
            st = list(car)
            for u in range(un):
                o = i * un + u
                for bg in range(_NBG):
                    x = buf[o, pl.ds(bg * _L, _L)]
                    nm = jnp.maximum(st[bg], x)
                    st[_NBG + bg] = jnp.where(
                        x > st[bg], zi + (off + o), st[_NBG + bg])
                    st[bg] = nm
            return tuple(st)

        car = lax.fori_loop(
            0, nv // un, p1,
            tuple(m0) + tuple(ist[pl.ds(bg * _L, _L)] for bg in range(_NBG)))
        m1 = car[:_NBG]
        for bg in range(_NBG):
            mst[pl.ds(bg * _L, _L)] = m1[bg]
            ist[pl.ds(bg * _L, _L)] = car[_NBG + bg]

        s0 = [jnp.where(m0[bg] > _NEG_INF,
                        sst[pl.ds(bg * _L, _L)] * jnp.exp(m0[bg] - m1[bg]),
                        0.0)
              for bg in range(_NBG)]

        def p2(i, car):
            st = list(car)
            for u in range(un):
                o = i * un + u
                for bg in range(_NBG):
                    x = buf[o, pl.ds(bg * _L, _L)]
                    st[bg] = st[bg] + jnp.exp(x - m1[bg])
            return tuple(st)

        s2 = lax.fori_loop(0, nv // un, p2, tuple(s0))
        for bg in range(_NBG):
            sst[pl.ds(bg * _L, _L)] = s2[bg]

        for bg in range(_NBG):
            av = act_v[pl.ds(bg * _L, _L)]
            loc = av - off
            inb = (loc >= 0) & (loc < nv)
            gath = plsc.load_gather(
                buf, [jnp.clip(loc, 0, nv - 1), bg * _L + lane])
            gst[pl.ds(bg * _L, _L)] = jnp.where(
                inb, gath, gst[pl.ds(bg * _L, _L)])

    dma(0, buf_a, sem_a)
    dma(1, buf_b, sem_b)

    @pl.loop(0, 16, step=2)
    def main_loop(c):
        wait(buf_a, sem_a)
        process(buf_a, vb + c * _W, _W, _U1)

        @pl.when(c < 14)
        def _():
            dma(c + 2, buf_a, sem_a)

        @pl.when(c == 14)
        def _():
            @pl.when(w < 31)
            def _():
                dma(16, buf_a, sem_a)

            @pl.when(w == 31)
            def _():
                pltpu.async_copy(
                    lt.at[pl.ds(vb + 16 * _W, _TWR), :], buf_t, sem_t)

        wait(buf_b, sem_b)
        process(buf_b, vb + (c + 1) * _W, _W, _U1)

        @pl.when(c < 13)
        def _():
            dma(c + 3, buf_b, sem_b)

    @pl.when(w < 31)
    def _():
        wait(buf_a, sem_a)
        process(buf_a, vb + 16 * _W, _W, _U1)

    @pl.when(w == 31)
    def _():
        pltpu.make_async_copy(
            lt.at[pl.ds(0, _TWR), :], buf_t, sem_t).wait()
        process(buf_t, vb + 16 * _W, _TWR, _UT)

    pltpu.sync_copy(gst, g_out.at[w])
    pltpu.sync_copy(mst, m_out.at[w])
    pltpu.sync_copy(sst, s_out.at[w])
    pltpu.sync_copy(ist, i_out.at[w])


_sc_part = functools.partial(
    pl.kernel,
    out_type=(
        jax.ShapeDtypeStruct((_NW, _B), jnp.float32),
        jax.ShapeDtypeStruct((_NW, _B), jnp.float32),
        jax.ShapeDtypeStruct((_NW, _B), jnp.float32),
        jax.ShapeDtypeStruct((_NW, _B), jnp.int32),
    ),
    mesh=plsc.VectorSubcoreMesh(
        core_axis_name="c", subcore_axis_name="s",
        num_cores=2, num_subcores=16),
    compiler_params=pltpu.CompilerParams(needs_layout_passes=False),
    scratch_types=[
        pltpu.VMEM((_W, _B), jnp.float32),
        pltpu.VMEM((_W, _B), jnp.float32),
        pltpu.VMEM((_TWR, _B), jnp.float32),
        pltpu.VMEM((_B,), jnp.int32),
        pltpu.VMEM((_B,), jnp.float32),
        pltpu.VMEM((_B,), jnp.float32),
        pltpu.VMEM((_B,), jnp.float32),
        pltpu.VMEM((_B,), jnp.int32),
        pltpu.SemaphoreType.DMA,
        pltpu.SemaphoreType.DMA,
        pltpu.SemaphoreType.DMA,
    ],
)(_sc_body)


def _tc_merge(m_ref, s_ref, i_ref, g_ref, lp_ref, md_ref):
    m = m_ref[...]
    mx = jnp.max(m, axis=0, keepdims=True)
    stot = jnp.sum(s_ref[...] * jnp.exp(m - mx), axis=0, keepdims=True)
    g = jnp.sum(g_ref[...], axis=0, keepdims=True)
    lp_ref[...] = g - (mx + jnp.log(stot))
    md_ref[...] = jnp.min(jnp.where(m == mx, i_ref[...], _I32_BIG),
                          axis=0, keepdims=True)


def kernel(logits, actions):
    act = actions.reshape(_B).astype(jnp.int32)
    lt = logits.T
    g_out, m_out, s_out, i_out = _sc_part(lt, act)
    lp, md = pl.pallas_call(
        _tc_merge,
        out_shape=(jax.ShapeDtypeStruct((1, _B), jnp.float32),
                   jax.ShapeDtypeStruct((1, _B), jnp.int32)),
    )(m_out, s_out, i_out, g_out)
    return lp.reshape(_B, 1), md.reshape(_B, 1)


# speedup vs baseline: 1.0277x; 1.0277x over previous
import functools

import jax
import jax.numpy as jnp
import numpy as np
from jax import lax
from jax.experimental import pallas as pl
from jax.experimental.pallas import tpu as pltpu
from jax.experimental.pallas import tpu_sc as plsc

_B = 128
_V = 100000
_L = 16
_NW = 32
_NBG = _B // _L
_VW = 3128
_W = 184
_TWR = _V - 31 * _VW - 16 * _W
_U1 = 2
_UT = 2

_NEG_INF = float("-inf")
_I32_BIG = np.int32(2**31 - 1)


def _sc_body(lt, act, g_out, m_out, s_out, i_out,
             buf_a, buf_b, buf_t, act_v, gst, mst, sst, ist,
             sem_a, sem_b, sem_t):
    w = lax.axis_index("s") * 2 + lax.axis_index("c")
    vb = w * _VW
    lane = lax.iota(jnp.int32, _L)
    neg = jnp.full((_L,), _NEG_INF, jnp.float32)
    zf = jnp.zeros((_L,), jnp.float32)
    zi = jnp.zeros((_L,), jnp.int32)

    pltpu.sync_copy(act, act_v)
    for bg in range(_NBG):
        mst[pl.ds(bg * _L, _L)] = neg
        ist[pl.ds(bg * _L, _L)] = zi
        sst[pl.ds(bg * _L, _L)] = zf
        gst[pl.ds(bg * _L, _L)] = zf

    def dma(c, buf, sem):
        pltpu.async_copy(lt.at[pl.ds(vb + c * _W, _W), :], buf, sem)

    def wait(buf, sem):
        pltpu.make_async_copy(lt.at[pl.ds(0, _W), :], buf, sem).wait()

    def process(buf, off, nv, un):
        m0 = [mst[pl.ds(bg * _L, _L)] for bg in range(_NBG)]

        def p1(i, car):
            st = list(car)
            for u in range(un):
                o = i * un + u
                for bg in range(_NBG):
                    x = buf[o, pl.ds(bg * _L, _L)]
                    nm = jnp.maximum(st[bg], x)
                    st[_NBG + bg] = jnp.where(
                        x > st[bg], zi + (off + o), st[_NBG + bg])
                    st[bg] = nm
            return tuple(st)

        car = plsc.parallel_loop(
            0, nv // un, 1, unroll=1,
            carry=tuple(m0) + tuple(
                ist[pl.ds(bg * _L, _L)] for bg in range(_NBG)))(p1)
        m1 = car[:_NBG]
        for bg in range(_NBG):
            mst[pl.ds(bg * _L, _L)] = m1[bg]
            ist[pl.ds(bg * _L, _L)] = car[_NBG + bg]

        s0 = [jnp.where(m0[bg] > _NEG_INF,
                        sst[pl.ds(bg * _L, _L)] * jnp.exp(m0[bg] - m1[bg]),
                        0.0)
              for bg in range(_NBG)]

        def p2(i, car):
            st = list(car)
            for u in range(un):
                o = i * un + u
                for bg in range(_NBG):
                    x = buf[o, pl.ds(bg * _L, _L)]
                    st[bg] = st[bg] + jnp.exp(x - m1[bg])
            return tuple(st)

        s2 = plsc.parallel_loop(0, nv // un, 1, unroll=1, carry=tuple(s0))(p2)
        for bg in range(_NBG):
            sst[pl.ds(bg * _L, _L)] = s2[bg]

        for bg in range(_NBG):
            av = act_v[pl.ds(bg * _L, _L)]
            loc = av - off
            inb = (loc >= 0) & (loc < nv)
            gath = plsc.load_gather(
                buf, [jnp.clip(loc, 0, nv - 1), bg * _L + lane])
            gst[pl.ds(bg * _L, _L)] = jnp.where(
                inb, gath, gst[pl.ds(bg * _L, _L)])

    dma(0, buf_a, sem_a)
    dma(1, buf_b, sem_b)

    @pl.loop(0, 16, step=2)
    def main_loop(c):
        wait(buf_a, sem_a)
        process(buf_a, vb + c * _W, _W, _U1)

        @pl.when(c < 14)
        def _():
            dma(c + 2, buf_a, sem_a)

        @pl.when(c == 14)
        def _():
            @pl.when(w < 31)
            def _():
                dma(16, buf_a, sem_a)

            @pl.when(w == 31)
            def _():
                pltpu.async_copy(
                    lt.at[pl.ds(vb + 16 * _W, _TWR), :], buf_t, sem_t)

        wait(buf_b, sem_b)
        process(buf_b, vb + (c + 1) * _W, _W, _U1)

        @pl.when(c < 13)
        def _():
            dma(c + 3, buf_b, sem_b)

    @pl.when(w < 31)
    def _():
        wait(buf_a, sem_a)
        process(buf_a, vb + 16 * _W, _W, _U1)

    @pl.when(w == 31)
    def _():
        pltpu.make_async_copy(
            lt.at[pl.ds(0, _TWR), :], buf_t, sem_t).wait()
        process(buf_t, vb + 16 * _W, _TWR, _UT)

    pltpu.sync_copy(gst, g_out.at[w])
    pltpu.sync_copy(mst, m_out.at[w])
    pltpu.sync_copy(sst, s_out.at[w])
    pltpu.sync_copy(ist, i_out.at[w])


_sc_part = functools.partial(
    pl.kernel,
    out_type=(
        jax.ShapeDtypeStruct((_NW, _B), jnp.float32),
        jax.ShapeDtypeStruct((_NW, _B), jnp.float32),
        jax.ShapeDtypeStruct((_NW, _B), jnp.float32),
        jax.ShapeDtypeStruct((_NW, _B), jnp.int32),
    ),
    mesh=plsc.VectorSubcoreMesh(
        core_axis_name="c", subcore_axis_name="s",
        num_cores=2, num_subcores=16),
    compiler_params=pltpu.CompilerParams(needs_layout_passes=False),
    scratch_types=[
        pltpu.VMEM((_W, _B), jnp.float32),
        pltpu.VMEM((_W, _B), jnp.float32),
        pltpu.VMEM((_TWR, _B), jnp.float32),
        pltpu.VMEM((_B,), jnp.int32),
        pltpu.VMEM((_B,), jnp.float32),
        pltpu.VMEM((_B,), jnp.float32),
        pltpu.VMEM((_B,), jnp.float32),
        pltpu.VMEM((_B,), jnp.int32),
        pltpu.SemaphoreType.DMA,
        pltpu.SemaphoreType.DMA,
        pltpu.SemaphoreType.DMA,
    ],
)(_sc_body)


def _tc_merge(m_ref, s_ref, i_ref, g_ref, lp_ref, md_ref):
    m = m_ref[...]
    mx = jnp.max(m, axis=0, keepdims=True)
    stot = jnp.sum(s_ref[...] * jnp.exp(m - mx), axis=0, keepdims=True)
    g = jnp.sum(g_ref[...], axis=0, keepdims=True)
    lp_ref[...] = g - (mx + jnp.log(stot))
    md_ref[...] = jnp.min(jnp.where(m == mx, i_ref[...], _I32_BIG),
                          axis=0, keepdims=True)


def kernel(logits, actions):
    act = actions.reshape(_B).astype(jnp.int32)
    lt = logits.T
    g_out, m_out, s_out, i_out = _sc_part(lt, act)
    lp, md = pl.pallas_call(
        _tc_merge,
        out_shape=(jax.ShapeDtypeStruct((1, _B), jnp.float32),
                   jax.ShapeDtypeStruct((1, _B), jnp.int32)),
    )(m_out, s_out, i_out, g_out)
    return lp.reshape(_B, 1), md.reshape(_B, 1)
